# baseline (speedup 1.0000x reference)
import math

import jax
import jax.numpy as jnp
from jax import lax
from jax.experimental import pallas as pl
from jax.experimental.pallas import tpu as pltpu
from jax.experimental.pallas import tpu_sc as plsc

V = 100000
D = 64
CS = 8
NS = 20
B = 16384
SIGMA = 0.1

NC = 2
NSC = 16
NW = NC * NSC
BPW = B // NW
CHUNK = 16
NCHUNK = BPW // CHUNK
NJ = NS + 1
NDOT = CHUNK * NJ
NGRP = NDOT // 16


def _sc_eta_body(tgt_hbm, ctx_hbm, neg_hbm, rho_hbm, alpha_hbm, out_hbm,
                 tgt_idx, ctx_idx, neg_idx_v, t_rows, a_rows, n_rows,
                 part_t, eta_all, sems):
    wid = lax.axis_index("s") * NC + lax.axis_index("c")
    lanes = lax.iota(jnp.int32, 16)

    pltpu.sync_copy(tgt_hbm.at[pl.ds(wid * BPW, BPW)], tgt_idx)
    pltpu.sync_copy(ctx_hbm.at[pl.ds(wid * BPW * CS, BPW * CS)], ctx_idx)
    pltpu.sync_copy(neg_hbm.at[pl.ds(wid * BPW * NS, BPW * NS)], neg_idx_v)

    def issue(c, s):
        cp_a = pltpu.async_copy(
            alpha_hbm.at[ctx_idx.at[pl.ds(c * CHUNK * CS, CHUNK * CS)]],
            a_rows.at[s], sems.at[s, 0])
        cp_t = pltpu.async_copy(
            rho_hbm.at[tgt_idx.at[pl.ds(c * CHUNK, CHUNK)]],
            t_rows.at[s], sems.at[s, 1])
        cp_n = pltpu.async_copy(
            rho_hbm.at[neg_idx_v.at[pl.ds(c * CHUNK * NS, CHUNK * NS)]],
            n_rows.at[s], sems.at[s, 2])
        return cp_a, cp_t, cp_n

    def wait_set(s):
        pltpu.make_async_copy(
            alpha_hbm.at[ctx_idx.at[pl.ds(0, CHUNK * CS)]],
            a_rows.at[s], sems.at[s, 0]).wait()
        pltpu.make_async_copy(
            rho_hbm.at[tgt_idx.at[pl.ds(0, CHUNK)]],
            t_rows.at[s], sems.at[s, 1]).wait()
        pltpu.make_async_copy(
            rho_hbm.at[neg_idx_v.at[pl.ds(0, CHUNK * NS)]],
            n_rows.at[s], sems.at[s, 2]).wait()

    def compute(c, s):
        def b_body(b, _):
            acc = []
            for h in range(2):
                v = a_rows[s, b * CS + 0, pl.ds(h * 32, 32)]
                for cc in range(1, CS):
                    v = v + a_rows[s, b * CS + cc, pl.ds(h * 32, 32)]
                a0, a1 = plsc.unpack(v, format=plsc.PackFormat.INTERLEAVED)
                acc.append(a0)
                acc.append(a1)

            def dot16(x0, x1, x2, x3):
                return x0 * acc[0] + x1 * acc[1] + x2 * acc[2] + x3 * acc[3]

            def row_dot(row_bf16_halves):
                h0, h1 = row_bf16_halves
                x0, x1 = plsc.unpack(h0, format=plsc.PackFormat.INTERLEAVED)
                x2, x3 = plsc.unpack(h1, format=plsc.PackFormat.INTERLEAVED)
                return dot16(x0, x1, x2, x3)

            dotbase = b * NJ
            p = row_dot([t_rows[s, b, pl.ds(h * 32, 32)] for h in range(2)])
            plsc.store_scatter(part_t, [lanes * NDOT + dotbase], p)
            for j in range(NS):
                q = row_dot([n_rows[s, b * NS + j, pl.ds(h * 32, 32)]
                             for h in range(2)])
                plsc.store_scatter(
                    part_t, [lanes * NDOT + (dotbase + 1 + j)], q)
            return 0

        lax.fori_loop(0, CHUNK, b_body, 0, unroll=False)

        def g_body(g, _):
            v = part_t[pl.ds(g * 16, 16)]
            for r in range(1, 16):
                v = v + part_t[pl.ds(r * NDOT + g * 16, 16)]
            eta_all[pl.ds(c * NDOT + g * 16, 16)] = v
            return 0

        lax.fori_loop(0, NGRP, g_body, 0, unroll=False)

    issue(0, 0)

    def pair_body(pr, _):
        i0 = 2 * pr
        issue(i0 + 1, 1)
        wait_set(0)
        compute(i0, 0)

        @pl.when(i0 + 2 < NCHUNK)
        def _():
            issue(i0 + 2, 0)

        wait_set(1)
        compute(i0 + 1, 1)
        return 0

    lax.fori_loop(0, NCHUNK // 2, pair_body, 0, unroll=False)
    pltpu.sync_copy(eta_all, out_hbm.at[pl.ds(wid * BPW * NJ, BPW * NJ)])


_sc_etas = pl.kernel(
    _sc_eta_body,
    out_type=jax.ShapeDtypeStruct((B * NJ,), jnp.float32),
    mesh=plsc.VectorSubcoreMesh(core_axis_name="c", subcore_axis_name="s",
                                num_cores=NC, num_subcores=NSC),
    compiler_params=pltpu.CompilerParams(needs_layout_passes=False,
                                         use_tc_tiling_on_sc=False),
    scratch_types=[
        pltpu.VMEM((BPW,), jnp.int32),
        pltpu.VMEM((BPW * CS,), jnp.int32),
        pltpu.VMEM((BPW * NS,), jnp.int32),
        pltpu.VMEM((2, CHUNK, D), jnp.bfloat16),
        pltpu.VMEM((2, CHUNK * CS, D), jnp.bfloat16),
        pltpu.VMEM((2, CHUNK * NS, D), jnp.bfloat16),
        pltpu.VMEM((16 * NDOT,), jnp.float32),
        pltpu.VMEM((BPW * NJ,), jnp.float32),
        pltpu.SemaphoreType.DMA((2, 3)),
    ],
)


_PRIOR_CONST = 2.0 * V * D * (-math.log(SIGMA) - 0.5 * math.log(2.0 * math.pi))

PRIOR_COLS = 8192
PRIOR_STEPS = -(-V // PRIOR_COLS)


def _tc_prior_body(rho_t_ref, alpha_t_ref, out_ref, acc_ref):
    step = pl.program_id(0)

    @pl.when(step == 0)
    def _():
        acc_ref[0] = 0.0

    col = step * PRIOR_COLS + lax.broadcasted_iota(
        jnp.int32, (D, PRIOR_COLS), 1)
    mask = col < V
    r = jnp.where(mask, rho_t_ref[...], 0.0)
    a = jnp.where(mask, alpha_t_ref[...], 0.0)
    acc_ref[0] += jnp.sum(r * r) + jnp.sum(a * a)

    @pl.when(step == PRIOR_STEPS - 1)
    def _():
        out_ref[0, 0] = acc_ref[0]


_tc_prior = pl.pallas_call(
    _tc_prior_body,
    grid=(PRIOR_STEPS,),
    in_specs=[
        pl.BlockSpec((D, PRIOR_COLS), lambda i: (0, i)),
        pl.BlockSpec((D, PRIOR_COLS), lambda i: (0, i)),
    ],
    out_specs=pl.BlockSpec(memory_space=pltpu.SMEM),
    out_shape=jax.ShapeDtypeStruct((1, 1), jnp.float32),
    scratch_shapes=[pltpu.SMEM((1,), jnp.float32)],
)


ETA_ROWS = (B * NJ) // 1024


def _tc_logsig_body(eta_ref, out_ref):
    eta = eta_ref[...]
    row = lax.broadcasted_iota(jnp.int32, eta.shape, 0)
    col = lax.broadcasted_iota(jnp.int32, eta.shape, 1)
    flat = row * 1024 + col
    signed = jnp.where(flat % NJ == 0, eta, -eta)
    out_ref[0, 0] = jnp.sum(jax.nn.log_sigmoid(signed))


_tc_logsig = pl.pallas_call(
    _tc_logsig_body,
    out_specs=pl.BlockSpec(memory_space=pltpu.SMEM),
    out_shape=jax.ShapeDtypeStruct((1, 1), jnp.float32),
)


def kernel(targets, contexts, neg_idx, rho, alpha):
    ctx_flat = contexts.reshape(-1)
    neg_flat = neg_idx.reshape(-1)
    eta = _sc_etas(targets, ctx_flat, neg_flat,
                   rho.astype(jnp.bfloat16), alpha.astype(jnp.bfloat16))
    sq = _tc_prior(rho.T, alpha.T)[0, 0]
    ll = _tc_logsig(eta.reshape(ETA_ROWS, 1024))[0, 0]
    return -(ll + (-0.5 / (SIGMA * SIGMA)) * sq + _PRIOR_CONST)
